# baseline (speedup 1.0000x reference)
import jax
import jax.numpy as jnp
from jax.experimental import pallas as pl
from jax.experimental.pallas import tpu as pltpu

N, C, H, W = 4, 96, 384, 384
G = 32
CG = C // G
EPS = 1e-5
NUM_IDS = 8
BH = 48


def _gn_kernel(ids_ref, x_ref, idx_ref, gamma_ref, beta_ref, out_ref):
    n = pl.program_id(0) // (H // BH)
    xb = x_ref[0]
    xg = xb.reshape(G, CG, BH, W)
    mean = jnp.mean(xg, axis=1, keepdims=True)
    diff = xg - mean
    var = jnp.mean(diff * diff, axis=1, keepdims=True)
    xnorm = (diff * jax.lax.rsqrt(var + EPS)).reshape(C, BH, W)
    gamma = gamma_ref[...][:, :, None]
    beta = beta_ref[...][:, :, None]
    xnorm = xnorm * gamma + beta
    idxb = idx_ref[0]
    mask = idxb == ids_ref[n, 0]
    for i in range(1, NUM_IDS):
        mask = mask | (idxb == ids_ref[n, i])
    out_ref[0] = jnp.where(mask[None, :, :], xnorm, xb)


def kernel(x, ins_indices_batch, ins_ids_list, gamma, beta):
    gamma2 = gamma.reshape(C, 1)
    beta2 = beta.reshape(C, 1)
    hb = H // BH
    grid = (N * hb,)
    out = pl.pallas_call(
        _gn_kernel,
        grid=grid,
        in_specs=[
            pl.BlockSpec(memory_space=pltpu.SMEM),
            pl.BlockSpec((1, C, BH, W), lambda i: (i // hb, 0, i % hb, 0)),
            pl.BlockSpec((1, BH, W), lambda i: (i // hb, i % hb, 0)),
            pl.BlockSpec((C, 1), lambda i: (0, 0)),
            pl.BlockSpec((C, 1), lambda i: (0, 0)),
        ],
        out_specs=pl.BlockSpec((1, C, BH, W), lambda i: (i // hb, 0, i % hb, 0)),
        out_shape=jax.ShapeDtypeStruct((N, C, H, W), x.dtype),
        compiler_params=pltpu.CompilerParams(
            dimension_semantics=("parallel",),
        ),
    )(ins_ids_list, x, ins_indices_batch, gamma2, beta2)
    return out


# speedup vs baseline: 1.0437x; 1.0437x over previous
import jax
import jax.numpy as jnp
from jax.experimental import pallas as pl
from jax.experimental.pallas import tpu as pltpu

N, C, H, W = 4, 96, 384, 384
G = 32
CG = C // G
EPS = 1e-5
NUM_IDS = 8
BH = 48
HB = H // BH
THIRD = 1.0 / 3.0


def _gn_kernel(ids_ref, gamma_ref, beta_ref, x_ref, idx_ref, out_ref):
    n = pl.program_id(0) // HB
    idxb = idx_ref[0]
    mask = idxb == ids_ref[n, 0]
    for i in range(1, NUM_IDS):
        mask = mask | (idxb == ids_ref[n, i])
    for g in range(G):
        a = x_ref[0, CG * g]
        b = x_ref[0, CG * g + 1]
        c = x_ref[0, CG * g + 2]
        m = (a + b + c) * THIRD
        da = a - m
        db = b - m
        dc = c - m
        var = (da * da + db * db + dc * dc) * THIRD
        rs = jax.lax.rsqrt(var + EPS)
        for k, d, orig in ((0, da, a), (1, db, b), (2, dc, c)):
            ch = CG * g + k
            y = d * rs * gamma_ref[ch] + beta_ref[ch]
            out_ref[0, ch] = jnp.where(mask, y, orig)


def kernel(x, ins_indices_batch, ins_ids_list, gamma, beta):
    grid = (N * HB,)
    out = pl.pallas_call(
        _gn_kernel,
        grid=grid,
        in_specs=[
            pl.BlockSpec(memory_space=pltpu.SMEM),
            pl.BlockSpec(memory_space=pltpu.SMEM),
            pl.BlockSpec(memory_space=pltpu.SMEM),
            pl.BlockSpec((1, C, BH, W), lambda i: (i // HB, 0, i % HB, 0)),
            pl.BlockSpec((1, BH, W), lambda i: (i // HB, i % HB, 0)),
        ],
        out_specs=pl.BlockSpec((1, C, BH, W), lambda i: (i // HB, 0, i % HB, 0)),
        out_shape=jax.ShapeDtypeStruct((N, C, H, W), x.dtype),
        compiler_params=pltpu.CompilerParams(
            dimension_semantics=("parallel",),
        ),
    )(ins_ids_list, gamma, beta, x, ins_indices_batch)
    return out


# speedup vs baseline: 1.0478x; 1.0039x over previous
import jax
import jax.numpy as jnp
from jax.experimental import pallas as pl
from jax.experimental.pallas import tpu as pltpu

N, C, H, W = 4, 96, 384, 384
G = 32
CG = C // G
EPS = 1e-5
NUM_IDS = 8
BH = 64
HB = H // BH
THIRD = 1.0 / 3.0


def _gn_kernel(ids_ref, gamma_ref, beta_ref, x_ref, idx_ref, out_ref):
    n = pl.program_id(0) // HB
    idxb = idx_ref[0]
    mask = idxb == ids_ref[n, 0]
    for i in range(1, NUM_IDS):
        mask = mask | (idxb == ids_ref[n, i])
    for g in range(G):
        a = x_ref[0, CG * g]
        b = x_ref[0, CG * g + 1]
        c = x_ref[0, CG * g + 2]
        m = (a + b + c) * THIRD
        da = a - m
        db = b - m
        dc = c - m
        var = (da * da + db * db + dc * dc) * THIRD
        rs = jax.lax.rsqrt(var + EPS)
        for k, d, orig in ((0, da, a), (1, db, b), (2, dc, c)):
            ch = CG * g + k
            y = d * rs * gamma_ref[ch] + beta_ref[ch]
            out_ref[0, ch] = jnp.where(mask, y, orig)


def kernel(x, ins_indices_batch, ins_ids_list, gamma, beta):
    grid = (N * HB,)
    out = pl.pallas_call(
        _gn_kernel,
        grid=grid,
        in_specs=[
            pl.BlockSpec(memory_space=pltpu.SMEM),
            pl.BlockSpec(memory_space=pltpu.SMEM),
            pl.BlockSpec(memory_space=pltpu.SMEM),
            pl.BlockSpec((1, C, BH, W), lambda i: (i // HB, 0, i % HB, 0)),
            pl.BlockSpec((1, BH, W), lambda i: (i // HB, i % HB, 0)),
        ],
        out_specs=pl.BlockSpec((1, C, BH, W), lambda i: (i // HB, 0, i % HB, 0)),
        out_shape=jax.ShapeDtypeStruct((N, C, H, W), x.dtype),
        compiler_params=pltpu.CompilerParams(
            dimension_semantics=("parallel",),
        ),
    )(ins_ids_list, gamma, beta, x, ins_indices_batch)
    return out
